# baseline (speedup 1.0000x reference)
import functools

import jax
import jax.numpy as jnp
from jax import lax
from jax.experimental import pallas as pl
from jax.experimental.pallas import tpu as pltpu
from jax.experimental.pallas import tpu_sc as plsc

E = 64
H = 768
F = 3072
N = 8192
T = 128
NT = N // T + E
NP = NT * T
TB = 1024
G = N // TB


def _gate_body(x_ref, wg_ref, bg_ref, eid_ref, psum_ref, cnt_ref, rib_ref):
    xb = x_ref[...]
    logits = jnp.dot(xb, wg_ref[...], preferred_element_type=jnp.float32)
    logits = logits + bg_ref[...]
    m = jnp.max(logits, axis=-1, keepdims=True)
    ex = jnp.exp(logits - m)
    s = jnp.sum(ex, axis=-1, keepdims=True)
    probs = ex / s
    ids = lax.broadcasted_iota(jnp.int32, logits.shape, 1)
    eid = jnp.min(jnp.where(logits == m, ids, E), axis=-1, keepdims=True)
    eid_ref[...] = eid
    psum_ref[...] = jnp.sum(probs, axis=0, keepdims=True)[None]
    onehot = (ids == eid).astype(jnp.float32)
    cnt_ref[...] = jnp.sum(onehot, axis=0, keepdims=True)[None]
    row = lax.broadcasted_iota(jnp.int32, (TB, TB), 0)
    col = lax.broadcasted_iota(jnp.int32, (TB, TB), 1)
    tril = (row > col).astype(jnp.float32)
    csum = jnp.dot(tril, onehot, preferred_element_type=jnp.float32)
    rib = jnp.sum(csum * onehot, axis=1, keepdims=True)
    rib_ref[...] = rib.astype(jnp.int32)


def _gate_call(x, Wg, bg2):
    return pl.pallas_call(
        _gate_body,
        grid=(G,),
        in_specs=[
            pl.BlockSpec((TB, H), lambda i: (i, 0)),
            pl.BlockSpec((H, E), lambda i: (0, 0)),
            pl.BlockSpec((1, E), lambda i: (0, 0)),
        ],
        out_specs=[
            pl.BlockSpec((TB, 1), lambda i: (i, 0)),
            pl.BlockSpec((1, 1, E), lambda i: (i, 0, 0)),
            pl.BlockSpec((1, 1, E), lambda i: (i, 0, 0)),
            pl.BlockSpec((TB, 1), lambda i: (i, 0)),
        ],
        out_shape=[
            jax.ShapeDtypeStruct((N, 1), jnp.int32),
            jax.ShapeDtypeStruct((G, 1, E), jnp.float32),
            jax.ShapeDtypeStruct((G, 1, E), jnp.float32),
            jax.ShapeDtypeStruct((N, 1), jnp.int32),
        ],
    )(x, Wg, bg2)


def _route_metadata(eid, rib, blk_cnt, counts):
    ntiles = (counts + (T - 1)) // T
    cum_tiles = jnp.cumsum(ntiles)
    padded_start = (cum_tiles - ntiles) * T
    blk_base = (jnp.cumsum(blk_cnt, axis=0) - blk_cnt).astype(jnp.int32)
    tok = jnp.arange(N, dtype=jnp.int32)
    rank = rib + blk_base.reshape(-1)[(tok // TB) * E + eid]
    pos = padded_start[eid] + rank
    pad_idx = jnp.arange(NP, dtype=jnp.int32) & (N - 1)
    src_idx = pad_idx.at[pos].set(tok)
    tile_ids = jnp.arange(NT, dtype=jnp.int32)
    expert_tile = jnp.minimum(
        jnp.searchsorted(cum_tiles, tile_ids, side="right"), E - 1
    ).astype(jnp.int32)
    valid_tile = (tile_ids < cum_tiles[-1]).astype(jnp.int32)
    return src_idx, pos, expert_tile, valid_tile


def _expert_body(et_ref, vt_ref, xs_ref, w1_ref, b1_ref, w2_ref, b2_ref,
                 wg_ref, bg_ref, ys_ref):
    del et_ref
    i = pl.program_id(0)

    @pl.when(vt_ref[i] > 0)
    def _():
        xb = xs_ref[...]
        logits = jnp.dot(xb, wg_ref[...], preferred_element_type=jnp.float32)
        logits = logits + bg_ref[...]
        m = jnp.max(logits, axis=-1, keepdims=True)
        p = 1.0 / jnp.sum(jnp.exp(logits - m), axis=-1, keepdims=True)
        h = jnp.dot(xb, w1_ref[0], preferred_element_type=jnp.float32)
        h = jnp.maximum(h + b1_ref[0], 0.0)
        o = jnp.dot(h, w2_ref[0], preferred_element_type=jnp.float32)
        ys_ref[...] = (o + b2_ref[0]) * p


def _expert_call(expert_tile, valid_tile, xs, W1, b1, W2, b2, Wg, bg2):
    grid_spec = pltpu.PrefetchScalarGridSpec(
        num_scalar_prefetch=2,
        grid=(NT,),
        in_specs=[
            pl.BlockSpec((T, H), lambda i, et, vt: (i, 0)),
            pl.BlockSpec((1, H, F), lambda i, et, vt: (et[i], 0, 0)),
            pl.BlockSpec((1, 1, F), lambda i, et, vt: (et[i], 0, 0)),
            pl.BlockSpec((1, F, H), lambda i, et, vt: (et[i], 0, 0)),
            pl.BlockSpec((1, 1, H), lambda i, et, vt: (et[i], 0, 0)),
            pl.BlockSpec((H, E), lambda i, et, vt: (0, 0)),
            pl.BlockSpec((1, E), lambda i, et, vt: (0, 0)),
        ],
        out_specs=pl.BlockSpec((T, H), lambda i, et, vt: (i, 0)),
    )
    return pl.pallas_call(
        _expert_body,
        grid_spec=grid_spec,
        out_shape=jax.ShapeDtypeStruct((NP, H), jnp.float32),
        compiler_params=pltpu.CompilerParams(
            dimension_semantics=("arbitrary",),
        ),
    )(expert_tile, valid_tile, xs, W1, b1, W2, b2, Wg, bg2)


@functools.lru_cache(maxsize=None)
def _make_sc_gather(n_rows_out, d):
    info = plsc.get_sparse_core_info()
    _NC = info.num_cores
    _NW = info.num_cores * info.num_subcores
    b_per_w = n_rows_out // _NW
    ch = 64 if b_per_w % 64 == 0 else b_per_w
    n_ch = b_per_w // ch
    mesh = plsc.VectorSubcoreMesh(core_axis_name="c", subcore_axis_name="s")

    @functools.partial(
        pl.kernel,
        mesh=mesh,
        out_type=jax.ShapeDtypeStruct((n_rows_out, d), jnp.float32),
        scratch_types=[
            pltpu.VMEM((b_per_w,), jnp.int32),
            pltpu.VMEM((ch, d), jnp.float32),
            pltpu.SemaphoreType.DMA,
        ],
    )
    def k(table_hbm, idx_hbm, out_hbm, idx_v, rows_v, sem):
        wid = lax.axis_index("s") * _NC + lax.axis_index("c")
        base = wid * b_per_w
        pltpu.sync_copy(idx_hbm.at[pl.ds(base, b_per_w)], idx_v)
        for c in range(n_ch):
            pltpu.async_copy(
                table_hbm.at[idx_v.at[pl.ds(c * ch, ch)]], rows_v, sem
            ).wait()
            pltpu.sync_copy(rows_v, out_hbm.at[pl.ds(base + c * ch, ch)])

    return k


def _gather_rows(table, idx):
    return _make_sc_gather(idx.shape[0], table.shape[1])(table, idx)


def kernel(x, Wg, bg, W1, b1, W2, b2):
    bg2 = bg.reshape(1, E)
    eid2, psum, cnt, rib2 = _gate_call(x, Wg, bg2)
    eid = eid2[:, 0]
    rib = rib2[:, 0]
    probs_sum = psum.sum(axis=(0, 1))
    counts_f = cnt.sum(axis=(0, 1))
    aux = jnp.dot(probs_sum / N, counts_f / N) * E
    counts = counts_f.astype(jnp.int32)
    blk_cnt = cnt[:, 0, :]
    src_idx, pos, expert_tile, valid_tile = _route_metadata(
        eid, rib, blk_cnt, counts)
    xs = _gather_rows(x, src_idx)
    ys = _expert_call(expert_tile, valid_tile, xs, W1, b1[:, None, :], W2,
                      b2[:, None, :], Wg, bg2)
    y = _gather_rows(ys, pos)
    return y, aux


# speedup vs baseline: 1.0792x; 1.0792x over previous
import functools

import jax
import jax.numpy as jnp
from jax import lax
from jax.experimental import pallas as pl
from jax.experimental.pallas import tpu as pltpu
from jax.experimental.pallas import tpu_sc as plsc

E = 64
H = 768
F = 3072
N = 8192
T = 128
NT = N // T + E
NP = NT * T
TB = 1024
G = N // TB


def _gate_body(x_ref, wg_ref, bg_ref, eid_ref, psum_ref, cnt_ref, rib_ref):
    xb = x_ref[...]
    logits = jnp.dot(xb, wg_ref[...], preferred_element_type=jnp.float32)
    logits = logits + bg_ref[...]
    m = jnp.max(logits, axis=-1, keepdims=True)
    ex = jnp.exp(logits - m)
    s = jnp.sum(ex, axis=-1, keepdims=True)
    probs = ex / s
    ids = lax.broadcasted_iota(jnp.int32, logits.shape, 1)
    eid = jnp.min(jnp.where(logits == m, ids, E), axis=-1, keepdims=True)
    eid_ref[...] = eid
    psum_ref[...] = jnp.sum(probs, axis=0, keepdims=True)[None]
    onehot = (ids == eid).astype(jnp.float32)
    cnt_ref[...] = jnp.sum(onehot, axis=0, keepdims=True)[None]
    row = lax.broadcasted_iota(jnp.int32, (TB, TB), 0)
    col = lax.broadcasted_iota(jnp.int32, (TB, TB), 1)
    tril = (row > col).astype(jnp.float32)
    csum = jnp.dot(tril, onehot, preferred_element_type=jnp.float32)
    rib = jnp.sum(csum * onehot, axis=1, keepdims=True)
    rib_ref[...] = rib.astype(jnp.int32)


def _gate_call(x, Wg, bg2):
    return pl.pallas_call(
        _gate_body,
        grid=(G,),
        in_specs=[
            pl.BlockSpec((TB, H), lambda i: (i, 0)),
            pl.BlockSpec((H, E), lambda i: (0, 0)),
            pl.BlockSpec((1, E), lambda i: (0, 0)),
        ],
        out_specs=[
            pl.BlockSpec((TB, 1), lambda i: (i, 0)),
            pl.BlockSpec((1, 1, E), lambda i: (i, 0, 0)),
            pl.BlockSpec((1, 1, E), lambda i: (i, 0, 0)),
            pl.BlockSpec((TB, 1), lambda i: (i, 0)),
        ],
        out_shape=[
            jax.ShapeDtypeStruct((N, 1), jnp.int32),
            jax.ShapeDtypeStruct((G, 1, E), jnp.float32),
            jax.ShapeDtypeStruct((G, 1, E), jnp.float32),
            jax.ShapeDtypeStruct((N, 1), jnp.int32),
        ],
    )(x, Wg, bg2)


def _route_metadata(eid, rib, blk_cnt, counts):
    ntiles = (counts + (T - 1)) // T
    cum_tiles = jnp.cumsum(ntiles)
    padded_start = (cum_tiles - ntiles) * T
    blk_base = (jnp.cumsum(blk_cnt, axis=0) - blk_cnt).astype(jnp.int32)
    tok = jnp.arange(N, dtype=jnp.int32)
    rank = rib + blk_base.reshape(-1)[(tok // TB) * E + eid]
    pos = padded_start[eid] + rank
    tile_ids = jnp.arange(NT, dtype=jnp.int32)
    expert_tile = jnp.minimum(
        jnp.searchsorted(cum_tiles, tile_ids, side="right"), E - 1
    ).astype(jnp.int32)
    valid_tile = (tile_ids < cum_tiles[-1]).astype(jnp.int32)
    return pos, expert_tile, valid_tile


def _expert_body(et_ref, vt_ref, xs_ref, w1_ref, b1_ref, w2_ref, b2_ref,
                 wg_ref, bg_ref, ys_ref):
    del et_ref
    i = pl.program_id(0)

    @pl.when(vt_ref[i] > 0)
    def _():
        xb = xs_ref[...]
        logits = jnp.dot(xb, wg_ref[...], preferred_element_type=jnp.float32)
        logits = logits + bg_ref[...]
        m = jnp.max(logits, axis=-1, keepdims=True)
        p = 1.0 / jnp.sum(jnp.exp(logits - m), axis=-1, keepdims=True)
        h = jnp.dot(xb, w1_ref[0], preferred_element_type=jnp.float32)
        h = jnp.maximum(h + b1_ref[0], 0.0)
        o = jnp.dot(h, w2_ref[0], preferred_element_type=jnp.float32)
        ys_ref[...] = (o + b2_ref[0]) * p


def _expert_call(expert_tile, valid_tile, xs, W1, b1, W2, b2, Wg, bg2):
    grid_spec = pltpu.PrefetchScalarGridSpec(
        num_scalar_prefetch=2,
        grid=(NT,),
        in_specs=[
            pl.BlockSpec((T, H), lambda i, et, vt: (i, 0)),
            pl.BlockSpec((1, H, F), lambda i, et, vt: (et[i], 0, 0)),
            pl.BlockSpec((1, 1, F), lambda i, et, vt: (et[i], 0, 0)),
            pl.BlockSpec((1, F, H), lambda i, et, vt: (et[i], 0, 0)),
            pl.BlockSpec((1, 1, H), lambda i, et, vt: (et[i], 0, 0)),
            pl.BlockSpec((H, E), lambda i, et, vt: (0, 0)),
            pl.BlockSpec((1, E), lambda i, et, vt: (0, 0)),
        ],
        out_specs=pl.BlockSpec((T, H), lambda i, et, vt: (i, 0)),
    )
    return pl.pallas_call(
        _expert_body,
        grid_spec=grid_spec,
        out_shape=jax.ShapeDtypeStruct((NP, H), jnp.float32),
        compiler_params=pltpu.CompilerParams(
            dimension_semantics=("arbitrary",),
        ),
    )(expert_tile, valid_tile, xs, W1, b1, W2, b2, Wg, bg2)


@functools.lru_cache(maxsize=None)
def _make_sc_gather(n_rows_out, d):
    info = plsc.get_sparse_core_info()
    _NC = info.num_cores
    _NW = info.num_cores * info.num_subcores
    b_per_w = n_rows_out // _NW
    ch = 64 if b_per_w % 64 == 0 else b_per_w
    n_ch = b_per_w // ch
    mesh = plsc.VectorSubcoreMesh(core_axis_name="c", subcore_axis_name="s")

    @functools.partial(
        pl.kernel,
        mesh=mesh,
        out_type=jax.ShapeDtypeStruct((n_rows_out, d), jnp.float32),
        scratch_types=[
            pltpu.VMEM((b_per_w,), jnp.int32),
            pltpu.VMEM((ch, d), jnp.float32),
            pltpu.SemaphoreType.DMA,
        ],
    )
    def k(table_hbm, idx_hbm, out_hbm, idx_v, rows_v, sem):
        wid = lax.axis_index("s") * _NC + lax.axis_index("c")
        base = wid * b_per_w
        pltpu.sync_copy(idx_hbm.at[pl.ds(base, b_per_w)], idx_v)
        for c in range(n_ch):
            pltpu.async_copy(
                table_hbm.at[idx_v.at[pl.ds(c * ch, ch)]], rows_v, sem
            ).wait()
            pltpu.sync_copy(rows_v, out_hbm.at[pl.ds(base + c * ch, ch)])

    return k


def _gather_rows(table, idx):
    return _make_sc_gather(idx.shape[0], table.shape[1])(table, idx)


@functools.lru_cache(maxsize=None)
def _make_sc_scatter(n_rows_in, n_rows_out, d):
    info = plsc.get_sparse_core_info()
    _NC = info.num_cores
    _NW = info.num_cores * info.num_subcores
    b_per_w = n_rows_in // _NW
    ch = 64 if b_per_w % 64 == 0 else b_per_w
    n_ch = b_per_w // ch
    mesh = plsc.VectorSubcoreMesh(core_axis_name="c", subcore_axis_name="s")

    @functools.partial(
        pl.kernel,
        mesh=mesh,
        out_type=jax.ShapeDtypeStruct((n_rows_out, d), jnp.float32),
        scratch_types=[
            pltpu.VMEM((n_ch, ch), jnp.int32),
            pltpu.VMEM((ch, d), jnp.float32),
            pltpu.SemaphoreType.DMA,
        ],
    )
    def k(src_hbm, idx_hbm, out_hbm, idx_v, rows_v, sem):
        wid = lax.axis_index("s") * _NC + lax.axis_index("c")
        base = wid * b_per_w
        pltpu.sync_copy(idx_hbm.at[wid], idx_v)
        for c in range(n_ch):
            pltpu.sync_copy(src_hbm.at[pl.ds(base + c * ch, ch)], rows_v)
            pltpu.async_copy(rows_v, out_hbm.at[idx_v.at[c]], sem).wait()

    return k


def _scatter_rows(src, idx, n_rows_out):
    info = plsc.get_sparse_core_info()
    nw = info.num_cores * info.num_subcores
    b_per_w = src.shape[0] // nw
    ch = 64 if b_per_w % 64 == 0 else b_per_w
    idx3 = idx.reshape(nw, b_per_w // ch, ch)
    return _make_sc_scatter(src.shape[0], n_rows_out, src.shape[1])(src, idx3)


def kernel(x, Wg, bg, W1, b1, W2, b2):
    bg2 = bg.reshape(1, E)
    eid2, psum, cnt, rib2 = _gate_call(x, Wg, bg2)
    eid = eid2[:, 0]
    rib = rib2[:, 0]
    probs_sum = psum.sum(axis=(0, 1))
    counts_f = cnt.sum(axis=(0, 1))
    aux = jnp.dot(probs_sum / N, counts_f / N) * E
    counts = counts_f.astype(jnp.int32)
    blk_cnt = cnt[:, 0, :]
    pos, expert_tile, valid_tile = _route_metadata(eid, rib, blk_cnt, counts)
    xs = _scatter_rows(x, pos, NP)
    ys = _expert_call(expert_tile, valid_tile, xs, W1, b1[:, None, :], W2,
                      b2[:, None, :], Wg, bg2)
    y = _gather_rows(ys, pos)
    return y, aux


# speedup vs baseline: 1.3813x; 1.2799x over previous
import functools

import jax
import jax.numpy as jnp
from jax import lax
from jax.experimental import pallas as pl
from jax.experimental.pallas import tpu as pltpu
from jax.experimental.pallas import tpu_sc as plsc

E = 64
H = 768
F = 3072
N = 8192
T = 128
NT = N // T + E
NP = NT * T
TB = 1024
G = N // TB


def _gate_body(x_ref, wg_ref, bg_ref, eid_ref, psum_ref, cnt_ref, rib_ref):
    xb = x_ref[...]
    logits = jnp.dot(xb, wg_ref[...], preferred_element_type=jnp.float32)
    logits = logits + bg_ref[...]
    m = jnp.max(logits, axis=-1, keepdims=True)
    ex = jnp.exp(logits - m)
    s = jnp.sum(ex, axis=-1, keepdims=True)
    probs = ex / s
    ids = lax.broadcasted_iota(jnp.int32, logits.shape, 1)
    eid = jnp.min(jnp.where(logits == m, ids, E), axis=-1, keepdims=True)
    eid_ref[...] = eid
    psum_ref[...] = jnp.sum(probs, axis=0, keepdims=True)[None]
    onehot = (ids == eid).astype(jnp.float32)
    cnt_ref[...] = jnp.sum(onehot, axis=0, keepdims=True)[None]
    row = lax.broadcasted_iota(jnp.int32, (TB, TB), 0)
    col = lax.broadcasted_iota(jnp.int32, (TB, TB), 1)
    tril = (row > col).astype(jnp.float32)
    csum = jnp.dot(tril, onehot, preferred_element_type=jnp.float32)
    rib = jnp.sum(csum * onehot, axis=1, keepdims=True)
    rib_ref[...] = rib.astype(jnp.int32)


def _gate_call(x, Wg, bg2):
    return pl.pallas_call(
        _gate_body,
        grid=(G,),
        in_specs=[
            pl.BlockSpec((TB, H), lambda i: (i, 0)),
            pl.BlockSpec((H, E), lambda i: (0, 0)),
            pl.BlockSpec((1, E), lambda i: (0, 0)),
        ],
        out_specs=[
            pl.BlockSpec((TB, 1), lambda i: (i, 0)),
            pl.BlockSpec((1, 1, E), lambda i: (i, 0, 0)),
            pl.BlockSpec((1, 1, E), lambda i: (i, 0, 0)),
            pl.BlockSpec((TB, 1), lambda i: (i, 0)),
        ],
        out_shape=[
            jax.ShapeDtypeStruct((N, 1), jnp.int32),
            jax.ShapeDtypeStruct((G, 1, E), jnp.float32),
            jax.ShapeDtypeStruct((G, 1, E), jnp.float32),
            jax.ShapeDtypeStruct((N, 1), jnp.int32),
        ],
    )(x, Wg, bg2)


def _route_body(eid_ref, rib_ref, cnt_ref, psum_ref,
                pos_ref, et_ref, vt_ref, aux_ref):
    i = pl.program_id(0)
    bc = cnt_ref[...][:, 0, :]
    counts = jnp.sum(bc, axis=0, keepdims=True)
    ntiles = jnp.floor((counts + (T - 1)) * (1.0 / T))
    er = lax.broadcasted_iota(jnp.int32, (E, E), 0)
    ec = lax.broadcasted_iota(jnp.int32, (E, E), 1)
    incl = (er <= ec).astype(jnp.float32)
    cum_tiles = jnp.dot(ntiles, incl, preferred_element_type=jnp.float32)
    padded_start = (cum_tiles - ntiles) * T
    gr = lax.broadcasted_iota(jnp.int32, (G, G), 0)
    gc = lax.broadcasted_iota(jnp.int32, (G, G), 1)
    trilG = (gr > gc).astype(jnp.float32)
    blk_base = jnp.dot(trilG, bc, preferred_element_type=jnp.float32)
    sel = (lax.broadcasted_iota(jnp.int32, (G, E), 0) == i).astype(jnp.float32)
    base_row = jnp.sum(blk_base * sel, axis=0, keepdims=True)
    eid = eid_ref[...]
    ids = lax.broadcasted_iota(jnp.int32, (TB, E), 1)
    oh = (ids == eid).astype(jnp.float32)
    pvec = padded_start + base_row
    pos_f = jnp.sum(oh * pvec, axis=1, keepdims=True)
    pos_ref[...] = pos_f.astype(jnp.int32) + rib_ref[...]
    tile_r = lax.broadcasted_iota(jnp.int32, (NT, 1), 0).astype(jnp.float32)
    n_le = jnp.sum((cum_tiles <= tile_r).astype(jnp.float32),
                   axis=1, keepdims=True)
    et_ref[...] = jnp.minimum(n_le, E - 1).astype(jnp.int32)
    total = jnp.sum(ntiles)
    vt_ref[...] = (tile_r < total).astype(jnp.int32)
    ps = jnp.sum(psum_ref[...][:, 0, :], axis=0, keepdims=True)
    aux_ref[...] = jnp.sum(ps * counts, axis=1, keepdims=True) * (
        float(E) / (float(N) * float(N)))


def _route_call(eid2, rib2, cnt, psum):
    return pl.pallas_call(
        _route_body,
        grid=(G,),
        in_specs=[
            pl.BlockSpec((TB, 1), lambda i: (i, 0)),
            pl.BlockSpec((TB, 1), lambda i: (i, 0)),
            pl.BlockSpec((G, 1, E), lambda i: (0, 0, 0)),
            pl.BlockSpec((G, 1, E), lambda i: (0, 0, 0)),
        ],
        out_specs=[
            pl.BlockSpec((TB, 1), lambda i: (i, 0)),
            pl.BlockSpec((NT, 1), lambda i: (0, 0)),
            pl.BlockSpec((NT, 1), lambda i: (0, 0)),
            pl.BlockSpec((1, 1), lambda i: (0, 0)),
        ],
        out_shape=[
            jax.ShapeDtypeStruct((N, 1), jnp.int32),
            jax.ShapeDtypeStruct((NT, 1), jnp.int32),
            jax.ShapeDtypeStruct((NT, 1), jnp.int32),
            jax.ShapeDtypeStruct((1, 1), jnp.float32),
        ],
    )(eid2, rib2, cnt, psum)


def _expert_body(et_ref, vt_ref, xs_ref, w1_ref, b1_ref, w2_ref, b2_ref,
                 wg_ref, bg_ref, ys_ref):
    del et_ref
    i = pl.program_id(0)

    @pl.when(vt_ref[i, 0] > 0)
    def _():
        xb = xs_ref[...]
        logits = jnp.dot(xb, wg_ref[...], preferred_element_type=jnp.float32)
        logits = logits + bg_ref[...]
        m = jnp.max(logits, axis=-1, keepdims=True)
        p = 1.0 / jnp.sum(jnp.exp(logits - m), axis=-1, keepdims=True)
        h = jnp.dot(xb, w1_ref[0], preferred_element_type=jnp.float32)
        h = jnp.maximum(h + b1_ref[0], 0.0)
        o = jnp.dot(h, w2_ref[0], preferred_element_type=jnp.float32)
        ys_ref[...] = (o + b2_ref[0]) * p


def _expert_call(expert_tile, valid_tile, xs, W1, b1, W2, b2, Wg, bg2):
    grid_spec = pltpu.PrefetchScalarGridSpec(
        num_scalar_prefetch=2,
        grid=(NT,),
        in_specs=[
            pl.BlockSpec((T, H), lambda i, et, vt: (i, 0)),
            pl.BlockSpec((1, H, F), lambda i, et, vt: (et[i, 0], 0, 0)),
            pl.BlockSpec((1, 1, F), lambda i, et, vt: (et[i, 0], 0, 0)),
            pl.BlockSpec((1, F, H), lambda i, et, vt: (et[i, 0], 0, 0)),
            pl.BlockSpec((1, 1, H), lambda i, et, vt: (et[i, 0], 0, 0)),
            pl.BlockSpec((H, E), lambda i, et, vt: (0, 0)),
            pl.BlockSpec((1, E), lambda i, et, vt: (0, 0)),
        ],
        out_specs=pl.BlockSpec((T, H), lambda i, et, vt: (i, 0)),
    )
    return pl.pallas_call(
        _expert_body,
        grid_spec=grid_spec,
        out_shape=jax.ShapeDtypeStruct((NP, H), jnp.float32),
        compiler_params=pltpu.CompilerParams(
            dimension_semantics=("arbitrary",),
        ),
    )(expert_tile, valid_tile, xs, W1, b1, W2, b2, Wg, bg2)


@functools.lru_cache(maxsize=None)
def _make_sc_gather(n_rows_out, d):
    info = plsc.get_sparse_core_info()
    _NC = info.num_cores
    _NW = info.num_cores * info.num_subcores
    b_per_w = n_rows_out // _NW
    ch = 64 if b_per_w % 64 == 0 else b_per_w
    n_ch = b_per_w // ch
    mesh = plsc.VectorSubcoreMesh(core_axis_name="c", subcore_axis_name="s")

    @functools.partial(
        pl.kernel,
        mesh=mesh,
        out_type=jax.ShapeDtypeStruct((n_rows_out, d), jnp.float32),
        scratch_types=[
            pltpu.VMEM((b_per_w,), jnp.int32),
            pltpu.VMEM((ch, d), jnp.float32),
            pltpu.SemaphoreType.DMA,
        ],
    )
    def k(table_hbm, idx_hbm, out_hbm, idx_v, rows_v, sem):
        wid = lax.axis_index("s") * _NC + lax.axis_index("c")
        base = wid * b_per_w
        pltpu.sync_copy(idx_hbm.at[pl.ds(base, b_per_w)], idx_v)
        for c in range(n_ch):
            pltpu.async_copy(
                table_hbm.at[idx_v.at[pl.ds(c * ch, ch)]], rows_v, sem
            ).wait()
            pltpu.sync_copy(rows_v, out_hbm.at[pl.ds(base + c * ch, ch)])

    return k


def _gather_rows(table, idx):
    return _make_sc_gather(idx.shape[0], table.shape[1])(table, idx)


@functools.lru_cache(maxsize=None)
def _make_sc_scatter(n_rows_in, n_rows_out, d):
    info = plsc.get_sparse_core_info()
    _NC = info.num_cores
    _NW = info.num_cores * info.num_subcores
    b_per_w = n_rows_in // _NW
    ch = 64 if b_per_w % 64 == 0 else b_per_w
    n_ch = b_per_w // ch
    mesh = plsc.VectorSubcoreMesh(core_axis_name="c", subcore_axis_name="s")

    @functools.partial(
        pl.kernel,
        mesh=mesh,
        out_type=jax.ShapeDtypeStruct((n_rows_out, d), jnp.float32),
        scratch_types=[
            pltpu.VMEM((n_ch, ch), jnp.int32),
            pltpu.VMEM((ch, d), jnp.float32),
            pltpu.SemaphoreType.DMA,
        ],
    )
    def k(src_hbm, idx_hbm, out_hbm, idx_v, rows_v, sem):
        wid = lax.axis_index("s") * _NC + lax.axis_index("c")
        base = wid * b_per_w
        pltpu.sync_copy(idx_hbm.at[wid], idx_v)
        for c in range(n_ch):
            pltpu.sync_copy(src_hbm.at[pl.ds(base + c * ch, ch)], rows_v)
            pltpu.async_copy(rows_v, out_hbm.at[idx_v.at[c]], sem).wait()

    return k


def _scatter_rows(src, idx, n_rows_out):
    info = plsc.get_sparse_core_info()
    nw = info.num_cores * info.num_subcores
    b_per_w = src.shape[0] // nw
    ch = 64 if b_per_w % 64 == 0 else b_per_w
    idx3 = idx.reshape(nw, b_per_w // ch, ch)
    return _make_sc_scatter(src.shape[0], n_rows_out, src.shape[1])(src, idx3)


def kernel(x, Wg, bg, W1, b1, W2, b2):
    bg2 = bg.reshape(1, E)
    eid2, psum, cnt, rib2 = _gate_call(x, Wg, bg2)
    pos2, expert_tile, valid_tile, aux11 = _route_call(eid2, rib2, cnt, psum)
    pos = pos2[:, 0]
    aux = aux11.reshape(())
    xs = _scatter_rows(x, pos, NP)
    ys = _expert_call(expert_tile, valid_tile, xs, W1, b1[:, None, :], W2,
                      b2[:, None, :], Wg, bg2)
    y = _gather_rows(ys, pos)
    return y, aux
